# initial kernel scaffold (unmeasured)
import jax
import jax.numpy as jnp
from jax import lax
from jax.experimental import pallas as pl
from jax.experimental.pallas import tpu as pltpu

N_DEV = 4
M_BLK = 1024
N_OUT = 8192
N_BLK = 1024
N_TILES = N_OUT // N_BLK
ORDER = (0, 1, 3, 2)
SEND_OFFSETS = (1, 3, 2)

_DevT = getattr(pl, "DeviceIdType", None) or pltpu.DeviceIdType
_CompilerParams = getattr(pltpu, "CompilerParams", None) or pltpu.TPUCompilerParams


def kernel(x, w_mat):
    k_total, m_shard = x.shape
    kw, n = w_mat.shape
    assert m_shard == M_BLK and k_total == N_DEV * M_BLK and n == N_OUT

    x_bf = x.astype(jnp.bfloat16)

    def body(x_hbm, w_hbm, out_ref, comm_ref, w_buf, amax_ref,
             local_sem, w_sems, send_sems, recv_sems,
             ax_send_sems, ax_recv_sems):
        my = lax.axis_index("i")

        barrier = pltpu.get_barrier_semaphore()
        for o in (1, 2, 3):
            pl.semaphore_signal(
                barrier, inc=1,
                device_id=((my + o) % N_DEV,),
                device_id_type=_DevT.MESH,
            )
        pl.semaphore_wait(barrier, N_DEV - 1)

        local_copy = pltpu.make_async_copy(
            x_hbm.at[pl.ds(my * M_BLK, M_BLK), :],
            comm_ref.at[pl.ds(0, M_BLK), :],
            local_sem,
        )
        local_copy.start()

        def send_desc(o):
            t = (my + o) % N_DEV
            r = N_DEV - o
            return pltpu.make_async_remote_copy(
                src_ref=x_hbm.at[pl.ds(t * M_BLK, M_BLK), :],
                dst_ref=comm_ref.at[pl.ds(r * M_BLK, M_BLK), :],
                send_sem=send_sems.at[o],
                recv_sem=recv_sems.at[r],
                device_id=(t,),
                device_id_type=_DevT.MESH,
            )

        for o in SEND_OFFSETS:
            send_desc(o).start()

        def w_copy(s, slot):
            k, j = divmod(s, N_TILES)
            cc = (my + ORDER[k]) % N_DEV
            return pltpu.make_async_copy(
                w_hbm.at[pl.ds(cc * M_BLK, M_BLK), pl.ds(j * N_BLK, N_BLK)],
                w_buf.at[slot],
                w_sems.at[slot],
            )

        def recv_desc(r):
            return pltpu.make_async_remote_copy(
                src_ref=x_hbm.at[pl.ds(0, M_BLK), :],
                dst_ref=comm_ref.at[pl.ds(r * M_BLK, M_BLK), :],
                send_sem=send_sems.at[0],
                recv_sem=recv_sems.at[r],
                device_id=(my,),
                device_id_type=_DevT.MESH,
            )

        n_steps = N_DEV * N_TILES
        w_copy(0, 0).start()
        w_copy(1, 1).start()

        for s in range(n_steps):
            k, j = divmod(s, N_TILES)
            slot = s % 2
            if j == 0:
                if k == 0:
                    local_copy.wait()
                else:
                    recv_desc(ORDER[k]).wait_recv()
            w_copy(s, slot).wait()
            a = comm_ref[pl.ds(ORDER[k] * M_BLK, M_BLK), :]
            b = w_buf[slot, :, :].astype(jnp.bfloat16)
            acc = jnp.dot(a, b, preferred_element_type=jnp.float32)
            ns = pl.ds(j * N_BLK, N_BLK)
            if k == 0:
                out_ref[:, ns] = acc
            else:
                out_ref[:, ns] = out_ref[:, ns] + acc
            if s + 2 < n_steps:
                w_copy(s + 2, slot).start()

        for o in SEND_OFFSETS:
            send_desc(o).wait_send()

        local_amax = jnp.max(jnp.abs(out_ref[:, :]))
        amax_ref[pl.ds(0, 8), :] = jnp.full((8, 128), local_amax, jnp.float32)

        def ax_desc(o):
            t = (my + o) % N_DEV
            r = N_DEV - o
            return pltpu.make_async_remote_copy(
                src_ref=amax_ref.at[pl.ds(0, 8), :],
                dst_ref=amax_ref.at[pl.ds(r * 8, 8), :],
                send_sem=ax_send_sems.at[o],
                recv_sem=ax_recv_sems.at[r],
                device_id=(t,),
                device_id_type=_DevT.MESH,
            )

        for o in (1, 2, 3):
            ax_desc(o).start()
        for o in (1, 2, 3):
            ax_desc(o).wait_recv()

        gmax = jnp.max(amax_ref[:, :])
        scale = gmax / 127.0

        for j in range(N_TILES):
            ns = pl.ds(j * N_BLK, N_BLK)
            q = jnp.clip(jnp.round(out_ref[:, ns] / scale), -127.0, 127.0)
            out_ref[:, ns] = q * scale

        for o in (1, 2, 3):
            ax_desc(o).wait_send()

    return pl.pallas_call(
        body,
        out_shape=jax.ShapeDtypeStruct((M_BLK, N_OUT), jnp.float32),
        in_specs=[
            pl.BlockSpec(memory_space=pltpu.ANY),
            pl.BlockSpec(memory_space=pltpu.ANY),
        ],
        out_specs=pl.BlockSpec(memory_space=pltpu.VMEM),
        scratch_shapes=[
            pltpu.VMEM((N_DEV * M_BLK, M_BLK), jnp.bfloat16),
            pltpu.VMEM((2, M_BLK, N_BLK), jnp.float32),
            pltpu.VMEM((N_DEV * 8, 128), jnp.float32),
            pltpu.SemaphoreType.DMA,
            pltpu.SemaphoreType.DMA((2,)),
            pltpu.SemaphoreType.DMA((4,)),
            pltpu.SemaphoreType.DMA((4,)),
            pltpu.SemaphoreType.DMA((4,)),
            pltpu.SemaphoreType.DMA((4,)),
        ],
        compiler_params=_CompilerParams(collective_id=0),
    )(x_bf, w_mat)


# baseline (device time: 150076 ns/iter reference)
import jax
import jax.numpy as jnp
from jax import lax
from jax.experimental import pallas as pl
from jax.experimental.pallas import tpu as pltpu

N_DEV = 4
M_BLK = 1024
N_OUT = 8192
N_BLK = 1024
N_TILES = N_OUT // N_BLK
ORDER = (0, 1, 3, 2)
SEND_OFFSETS = (1, 3, 2)

_DevT = getattr(pl, "DeviceIdType", None) or pltpu.DeviceIdType
_CompilerParams = getattr(pltpu, "CompilerParams", None) or pltpu.TPUCompilerParams


def kernel(x, w_mat):
    k_total, m_shard = x.shape
    kw, n = w_mat.shape
    assert m_shard == M_BLK and k_total == N_DEV * M_BLK and n == N_OUT

    x_bf = x.astype(jnp.bfloat16)

    def body(x_hbm, w_hbm, out_ref, comm_ref, w_buf, amax_ref,
             local_sem, w_sems, send_sems, recv_sems,
             ax_send_sems, ax_recv_sems):
        my = lax.axis_index("i")

        barrier = pltpu.get_barrier_semaphore()
        for o in (1, 2, 3):
            pl.semaphore_signal(
                barrier, inc=1,
                device_id=((my + o) % N_DEV,),
                device_id_type=_DevT.MESH,
            )
        pl.semaphore_wait(barrier, N_DEV - 1)

        local_copy = pltpu.make_async_copy(
            x_hbm.at[pl.ds(my * M_BLK, M_BLK), :],
            comm_ref.at[pl.ds(0, M_BLK), :],
            local_sem,
        )
        local_copy.start()

        def send_desc(o):
            t = (my + o) % N_DEV
            r = N_DEV - o
            return pltpu.make_async_remote_copy(
                src_ref=x_hbm.at[pl.ds(t * M_BLK, M_BLK), :],
                dst_ref=comm_ref.at[pl.ds(r * M_BLK, M_BLK), :],
                send_sem=send_sems.at[o],
                recv_sem=recv_sems.at[r],
                device_id=(t,),
                device_id_type=_DevT.MESH,
            )

        for o in SEND_OFFSETS:
            send_desc(o).start()

        def w_copy(s, slot):
            k, j = divmod(s, N_TILES)
            cc = (my + ORDER[k]) % N_DEV
            return pltpu.make_async_copy(
                w_hbm.at[pl.ds(cc * M_BLK, M_BLK), pl.ds(j * N_BLK, N_BLK)],
                w_buf.at[slot],
                w_sems.at[slot],
            )

        def recv_desc(r):
            return pltpu.make_async_remote_copy(
                src_ref=x_hbm.at[pl.ds(0, M_BLK), :],
                dst_ref=comm_ref.at[pl.ds(r * M_BLK, M_BLK), :],
                send_sem=send_sems.at[0],
                recv_sem=recv_sems.at[r],
                device_id=(my,),
                device_id_type=_DevT.MESH,
            )

        n_steps = N_DEV * N_TILES
        w_copy(0, 0).start()
        w_copy(1, 1).start()

        for s in range(n_steps):
            k, j = divmod(s, N_TILES)
            slot = s % 2
            if j == 0:
                if k == 0:
                    local_copy.wait()
                else:
                    recv_desc(ORDER[k]).wait_recv()
            w_copy(s, slot).wait()
            a = comm_ref[pl.ds(ORDER[k] * M_BLK, M_BLK), :]
            b = w_buf[slot, :, :].astype(jnp.bfloat16)
            acc = jnp.dot(a, b, preferred_element_type=jnp.float32)
            ns = pl.ds(j * N_BLK, N_BLK)
            if k == 0:
                out_ref[:, ns] = acc
            else:
                out_ref[:, ns] = out_ref[:, ns] + acc
            if s + 2 < n_steps:
                w_copy(s + 2, slot).start()

        for o in SEND_OFFSETS:
            send_desc(o).wait_send()

        local_amax = jnp.max(jnp.abs(out_ref[:, :]))
        amax_ref[pl.ds(0, 8), :] = jnp.full((8, 128), local_amax, jnp.float32)

        def ax_desc(o):
            t = (my + o) % N_DEV
            r = N_DEV - o
            return pltpu.make_async_remote_copy(
                src_ref=amax_ref.at[pl.ds(0, 8), :],
                dst_ref=amax_ref.at[pl.ds(r * 8, 8), :],
                send_sem=ax_send_sems.at[o],
                recv_sem=ax_recv_sems.at[r],
                device_id=(t,),
                device_id_type=_DevT.MESH,
            )

        for o in (1, 2, 3):
            ax_desc(o).start()
        for o in (1, 2, 3):
            ax_desc(o).wait_recv()

        gmax = jnp.max(amax_ref[:, :])
        scale = gmax / 127.0

        for j in range(N_TILES):
            ns = pl.ds(j * N_BLK, N_BLK)
            q = jnp.clip(jnp.round(out_ref[:, ns] / scale), -127.0, 127.0)
            out_ref[:, ns] = q * scale

        for o in (1, 2, 3):
            ax_desc(o).wait_send()

    return pl.pallas_call(
        body,
        out_shape=jax.ShapeDtypeStruct((M_BLK, N_OUT), jnp.float32),
        in_specs=[
            pl.BlockSpec(memory_space=pl.ANY),
            pl.BlockSpec(memory_space=pl.ANY),
        ],
        out_specs=pl.BlockSpec(memory_space=pltpu.VMEM),
        scratch_shapes=[
            pltpu.VMEM((N_DEV * M_BLK, M_BLK), jnp.bfloat16),
            pltpu.VMEM((2, M_BLK, N_BLK), jnp.float32),
            pltpu.VMEM((N_DEV * 8, 128), jnp.float32),
            pltpu.SemaphoreType.DMA,
            pltpu.SemaphoreType.DMA((2,)),
            pltpu.SemaphoreType.DMA((4,)),
            pltpu.SemaphoreType.DMA((4,)),
            pltpu.SemaphoreType.DMA((4,)),
            pltpu.SemaphoreType.DMA((4,)),
        ],
        compiler_params=_CompilerParams(
            collective_id=0,
            vmem_limit_bytes=100 * 1024 * 1024,
        ),
    )(x_bf, w_mat)


# device time: 147151 ns/iter; 1.0199x vs baseline; 1.0199x over previous
import jax
import jax.numpy as jnp
from jax import lax
from jax.experimental import pallas as pl
from jax.experimental.pallas import tpu as pltpu

N_DEV = 4
M_BLK = 1024
N_OUT = 8192
N_BLK = 1024
N_TILES = N_OUT // N_BLK
ORDER = (0, 1, 3, 2)
SEND_OFFSETS = (1, 3, 2)

_DevT = getattr(pl, "DeviceIdType", None) or pltpu.DeviceIdType
_CompilerParams = getattr(pltpu, "CompilerParams", None) or pltpu.TPUCompilerParams


def kernel(x, w_mat):
    k_total, m_shard = x.shape
    kw, n = w_mat.shape
    assert m_shard == M_BLK and k_total == N_DEV * M_BLK and n == N_OUT

    x_bf = x.astype(jnp.bfloat16)

    def body(x_hbm, w_hbm, out_ref, comm_ref, w_buf, amax_ref,
             local_sem, w_sems, send_sems, recv_sems,
             ax_send_sems, ax_recv_sems):
        my = lax.axis_index("i")

        barrier = pltpu.get_barrier_semaphore()
        for o in (1, 2, 3):
            pl.semaphore_signal(
                barrier, inc=1,
                device_id=((my + o) % N_DEV,),
                device_id_type=_DevT.MESH,
            )
        pl.semaphore_wait(barrier, N_DEV - 1)

        local_copy = pltpu.make_async_copy(
            x_hbm.at[pl.ds(my * M_BLK, M_BLK), :],
            comm_ref.at[pl.ds(0, M_BLK), :],
            local_sem,
        )
        local_copy.start()

        def send_desc(o):
            t = (my + o) % N_DEV
            r = N_DEV - o
            return pltpu.make_async_remote_copy(
                src_ref=x_hbm.at[pl.ds(t * M_BLK, M_BLK), :],
                dst_ref=comm_ref.at[pl.ds(r * M_BLK, M_BLK), :],
                send_sem=send_sems.at[o],
                recv_sem=recv_sems.at[r],
                device_id=(t,),
                device_id_type=_DevT.MESH,
            )

        for o in SEND_OFFSETS:
            send_desc(o).start()

        def w_copy(s, slot):
            k, j = divmod(s, N_TILES)
            cc = (my + ORDER[k]) % N_DEV
            return pltpu.make_async_copy(
                w_hbm.at[pl.ds(cc * M_BLK, M_BLK), pl.ds(j * N_BLK, N_BLK)],
                w_buf.at[slot],
                w_sems.at[slot],
            )

        def recv_desc(r):
            return pltpu.make_async_remote_copy(
                src_ref=x_hbm.at[pl.ds(0, M_BLK), :],
                dst_ref=comm_ref.at[pl.ds(r * M_BLK, M_BLK), :],
                send_sem=send_sems.at[0],
                recv_sem=recv_sems.at[r],
                device_id=(my,),
                device_id_type=_DevT.MESH,
            )

        n_steps = N_DEV * N_TILES
        w_copy(0, 0).start()
        w_copy(1, 1).start()

        tile_maxes = []
        for s in range(n_steps):
            k, j = divmod(s, N_TILES)
            slot = s % 2
            if j == 0:
                if k == 0:
                    local_copy.wait()
                else:
                    recv_desc(ORDER[k]).wait_recv()
            w_copy(s, slot).wait()
            a = comm_ref[pl.ds(ORDER[k] * M_BLK, M_BLK), :]
            b = w_buf[slot, :, :].astype(jnp.bfloat16)
            acc = jnp.dot(a, b, preferred_element_type=jnp.float32)
            ns = pl.ds(j * N_BLK, N_BLK)
            if k == 0:
                out_ref[:, ns] = acc
            elif k < N_DEV - 1:
                out_ref[:, ns] = out_ref[:, ns] + acc
            else:
                val = out_ref[:, ns] + acc
                out_ref[:, ns] = val
                tile_maxes.append(jnp.max(jnp.abs(val)))
            if s + 2 < n_steps:
                w_copy(s + 2, slot).start()

        for o in SEND_OFFSETS:
            send_desc(o).wait_send()

        local_amax = tile_maxes[0]
        for tm in tile_maxes[1:]:
            local_amax = jnp.maximum(local_amax, tm)
        amax_ref[pl.ds(0, 8), :] = jnp.full((8, 128), local_amax, jnp.float32)

        def ax_desc(o):
            t = (my + o) % N_DEV
            r = N_DEV - o
            return pltpu.make_async_remote_copy(
                src_ref=amax_ref.at[pl.ds(0, 8), :],
                dst_ref=amax_ref.at[pl.ds(r * 8, 8), :],
                send_sem=ax_send_sems.at[o],
                recv_sem=ax_recv_sems.at[r],
                device_id=(t,),
                device_id_type=_DevT.MESH,
            )

        for o in (1, 2, 3):
            ax_desc(o).start()
        for o in (1, 2, 3):
            ax_desc(o).wait_recv()

        gmax = jnp.max(amax_ref[:, :])
        scale = gmax / 127.0
        inv_scale = 127.0 / gmax

        for j in range(N_TILES):
            ns = pl.ds(j * N_BLK, N_BLK)
            q = jnp.clip(jnp.round(out_ref[:, ns] * inv_scale), -127.0, 127.0)
            out_ref[:, ns] = q * scale

        for o in (1, 2, 3):
            ax_desc(o).wait_send()

    return pl.pallas_call(
        body,
        out_shape=jax.ShapeDtypeStruct((M_BLK, N_OUT), jnp.float32),
        in_specs=[
            pl.BlockSpec(memory_space=pl.ANY),
            pl.BlockSpec(memory_space=pl.ANY),
        ],
        out_specs=pl.BlockSpec(memory_space=pltpu.VMEM),
        scratch_shapes=[
            pltpu.VMEM((N_DEV * M_BLK, M_BLK), jnp.bfloat16),
            pltpu.VMEM((2, M_BLK, N_BLK), jnp.float32),
            pltpu.VMEM((N_DEV * 8, 128), jnp.float32),
            pltpu.SemaphoreType.DMA,
            pltpu.SemaphoreType.DMA((2,)),
            pltpu.SemaphoreType.DMA((4,)),
            pltpu.SemaphoreType.DMA((4,)),
            pltpu.SemaphoreType.DMA((4,)),
            pltpu.SemaphoreType.DMA((4,)),
        ],
        compiler_params=_CompilerParams(
            collective_id=0,
            vmem_limit_bytes=100 * 1024 * 1024,
        ),
    )(x_bf, w_mat)
